# baseline (speedup 1.0000x reference)
import jax
import jax.numpy as jnp
from jax.experimental import pallas as pl

G = 1
B_TOT = 256
K = 1024
D = 64
HID = 128

_TH = (((0, 0), (2, 1)), ((1, 1), (3, 2)))
_RM = {0: (1, 0), 1: (0, 0), 2: (1, 1), 3: (0, 1)}


def _mm(a, b):
    return jax.lax.dot_general(a, b, (((1,), (0,)), ((), ())),
                               preferred_element_type=jnp.float32)


def _vqvae_block(p1_ref, w1_ref, b1_ref, w2t_ref, b2_ref, cb_ref,
                 d1t_ref, db1_ref, d2t_ref, db2_ref, *out_refs):
    g = p1_ref.shape[0]
    f32 = jnp.float32
    w1 = w1_ref[...]
    b1 = b1_ref[...]

    h1 = {}
    for py in range(2):
        for px in range(2):
            p = p1_ref[:, py * 2 + px]
            m = _mm(p.reshape(g * 64, 48), w1)
            h1[(py, px)] = jax.nn.relu(m + b1[None, :]).reshape(g, 8, 8, HID)

    def _pad4(arrs, n):
        out = {}
        for pr in range(2):
            for pc in range(2):
                a = arrs[(pr, pc)]
                out[(pr, pc)] = jnp.pad(
                    a, ((0, 0), (pr, 1 - pr), (pc, 1 - pc), (0, 0)))
        return out

    h1p = _pad4(h1, 9)

    w2t = w2t_ref[...]
    e = jnp.zeros((g * 64, D), f32)
    for kh in range(4):
        pr, oy = _RM[kh]
        for kw in range(4):
            pc, ox = _RM[kw]
            sl = h1p[(pr, pc)][:, oy:oy + 8, ox:ox + 8, :]
            e = e + _mm(sl.reshape(g * 64, HID), w2t[kh * 4 + kw])
    e = e + b2_ref[...][None, :]

    cb = cb_ref[...]
    cn = jnp.sum(cb * cb, axis=1)
    scores = cn[None, :] - 2.0 * jax.lax.dot_general(
        e, cb, (((1,), (1,)), ((), ())), preferred_element_type=f32)
    idx = jnp.argmin(scores, axis=1)
    onehot = (jax.lax.broadcasted_iota(jnp.int32, (g * 64, K), 1)
              == idx[:, None]).astype(f32)
    q = _mm(onehot, cb).reshape(g, 8, 8, D)

    qp = jnp.pad(q, ((0, 0), (1, 1), (1, 1), (0, 0)))
    d1t = d1t_ref[...]
    db1 = db1_ref[...]
    h2 = {}
    for r in range(2):
        for s in range(2):
            acc = jnp.zeros((g * 64, HID), f32)
            for (kh, dy) in _TH[r]:
                for (kw, dx) in _TH[s]:
                    sl = qp[:, dy:dy + 8, dx:dx + 8, :]
                    acc = acc + _mm(sl.reshape(g * 64, D), d1t[kh * 4 + kw])
            h2[(r, s)] = jax.nn.relu(acc + db1[None, :]).reshape(g, 8, 8, HID)

    h2p = {k: jnp.pad(v, ((0, 0), (1, 1), (1, 1), (0, 0)))
           for k, v in h2.items()}

    d2t = d2t_ref[...]
    db2 = db2_ref[...]
    for R in range(2):
        for ap in range(2):
            ty = 2 * ap + R
            for S in range(2):
                for bp in range(2):
                    tx = 2 * bp + S
                    acc = jnp.zeros((g * 64, 3), f32)
                    for (kh, dy) in _TH[R]:
                        cy = ap + dy - 1
                        ry, sy = cy & 1, cy >> 1
                        for (kw, dx) in _TH[S]:
                            cx = bp + dx - 1
                            rx, sx = cx & 1, cx >> 1
                            sl = h2p[(ry, rx)][:, 1 + sy:9 + sy,
                                               1 + sx:9 + sx, :]
                            acc = acc + _mm(sl.reshape(g * 64, HID),
                                            d2t[kh * 4 + kw])
                    out_refs[ty * 4 + tx][...] = \
                        (acc + db2[None, :]).reshape(g, 8, 8, 3)


def _run(p1p, w1r, b1, w2t, b2, cb, d1t, db1, d2t, db2, *, interpret=False):
    grid = (B_TOT // G,)
    full = lambda a: pl.BlockSpec(a.shape, lambda i: (0,) * a.ndim)
    blk = pl.BlockSpec((G, 4, 8, 8, 48), lambda i: (i, 0, 0, 0, 0))
    oblk = pl.BlockSpec((G, 8, 8, 3), lambda i: (i, 0, 0, 0))
    oshape = jax.ShapeDtypeStruct((B_TOT, 8, 8, 3), jnp.float32)
    return pl.pallas_call(
        _vqvae_block,
        grid=grid,
        in_specs=[blk, full(w1r), full(b1), full(w2t), full(b2), full(cb),
                  full(d1t), full(db1), full(d2t), full(db2)],
        out_specs=[oblk] * 16,
        out_shape=[oshape] * 16,
        interpret=interpret,
    )(p1p, w1r, b1, w2t, b2, cb, d1t, db1, d2t, db2)


def kernel(x, enc_w1, enc_b1, enc_w2, enc_b2, codebook,
           dec_w1, dec_b1, dec_w2, dec_b2):
    xn = jnp.transpose(x, (0, 2, 3, 1))
    xp = jnp.pad(xn, ((0, 0), (1, 1), (1, 1), (0, 0)))
    cols = [xp[:, kh:kh + 31:2, kw:kw + 31:2, :]
            for kh in range(4) for kw in range(4)]
    p1 = jnp.concatenate(cols, axis=-1)
    p1p = jnp.stack([p1[:, py::2, px::2, :]
                     for py in range(2) for px in range(2)], axis=1)

    w1r = enc_w1.transpose(2, 3, 1, 0).reshape(48, HID)
    w2t = enc_w2.transpose(2, 3, 1, 0).reshape(16, HID, D)
    d1t = dec_w1.transpose(2, 3, 1, 0).reshape(16, D, HID)
    d2t = dec_w2.transpose(2, 3, 1, 0).reshape(16, HID, 3)

    subs = _run(p1p, w1r, enc_b1, w2t, enc_b2, codebook,
                d1t, dec_b1, d2t, dec_b2)

    rows = [jnp.stack(subs[ty * 4:ty * 4 + 4], axis=3) for ty in range(4)]
    st = jnp.stack(rows, axis=2)
    out = st.reshape(B_TOT, 32, 32, 3)
    return jnp.transpose(out, (0, 3, 1, 2))


# speedup vs baseline: 10.7484x; 10.7484x over previous
import jax
import jax.numpy as jnp
from jax.experimental import pallas as pl

G = 16
B_TOT = 256
K = 1024
D = 64
HID = 128


def _mm(a, b):
    return jax.lax.dot_general(a, b, (((1,), (0,)), ((), ())),
                               preferred_element_type=jnp.float32)


def _vqvae_block(p1_ref, w1_ref, b1_ref, w2c_ref, b2_ref, cb_ref, cbt_ref,
                 d1t_ref, db1_ref, d2a_ref, db2_ref, *out_refs):
    g = p1_ref.shape[0]
    f32 = jnp.float32

    p1 = p1_ref[...]
    m = _mm(p1.reshape(g * 4 * 64, 48), w1_ref[...])
    h1 = jax.nn.relu(m + b1_ref[...][None, :]).reshape(g, 4, 8, 8, HID)

    h1p = {}
    for py in range(2):
        for px in range(2):
            h1p[(py, px)] = jnp.pad(
                h1[:, py * 2 + px],
                ((0, 0), (py, 1 - py), (px, 1 - px), (0, 0)))

    w2c = w2c_ref[...]
    e = jnp.zeros((g * 64, D), f32)
    for pr in range(2):
        for pc in range(2):
            a = h1p[(pr, pc)]
            cat = jnp.concatenate(
                [a[:, oy:oy + 8, ox:ox + 8, :]
                 for oy in range(2) for ox in range(2)], axis=-1)
            e = e + _mm(cat.reshape(g * 64, 4 * HID), w2c[pr * 2 + pc])
    e = e + b2_ref[...][None, :]

    cb = cb_ref[...]
    cbt = cbt_ref[...]
    cn = jnp.sum(cbt * cbt, axis=0)
    scores = cn[None, :] - 2.0 * _mm(e, cbt)
    idx = jnp.argmin(scores, axis=1)
    onehot = (jax.lax.broadcasted_iota(jnp.int32, (g * 64, K), 1)
              == idx[:, None]).astype(f32)
    q = _mm(onehot, cb)

    d1t = d1t_ref[...]
    db1 = db1_ref[...]
    big = _mm(q, d1t)
    pp1 = [jnp.pad(big[:, HID * t:HID * (t + 1)].reshape(g, 8, 8, HID),
                   ((0, 0), (1, 1), (1, 1), (0, 0)))
           for t in range(16)]
    h2 = {}
    for R in range(2):
        for S in range(2):
            acc = None
            for i in range(2):
                sy = R - 1 + i
                for j in range(2):
                    sx = S - 1 + j
                    t = (R + 2 * i) * 4 + (S + 2 * j)
                    sl = pp1[t][:, 1 + sy:9 + sy, 1 + sx:9 + sx, :]
                    acc = sl if acc is None else acc + sl
            h2[(R, S)] = jax.nn.relu(acc + db1[None, None, None, :])

    d2c = d2a_ref[...]
    db2 = db2_ref[...]
    h2p = {k: jnp.pad(v, ((0, 0), (1, 1), (1, 1), (0, 0)))
           for k, v in h2.items()}
    for ty in range(4):
        R, apar = ty & 1, ty >> 1
        for tx in range(4):
            S, bpar = tx & 1, tx >> 1
            parts = []
            for i in range(2):
                cy = apar + R - 1 + i
                ry, sy = cy & 1, cy >> 1
                for j in range(2):
                    cx = bpar + S - 1 + j
                    rx, sx = cx & 1, cx >> 1
                    parts.append(h2p[(ry, rx)][:, 1 + sy:9 + sy,
                                               1 + sx:9 + sx, :])
            cat = jnp.concatenate(parts, axis=-1)
            res = _mm(cat.reshape(g * 64, 4 * HID), d2c[ty * 4 + tx])
            out_refs[ty * 4 + tx][...] = \
                (res + db2[None, :]).reshape(g, 8, 8, 3)


def _run(p1p, w1r, b1, w2c, b2, cb, cbt, d1t, db1, d2a, db2, *,
         interpret=False):
    grid = (B_TOT // G,)
    full = lambda a: pl.BlockSpec(a.shape, lambda i: (0,) * a.ndim)
    blk = pl.BlockSpec((G, 4, 8, 8, 48), lambda i: (i, 0, 0, 0, 0))
    oblk = pl.BlockSpec((G, 8, 8, 3), lambda i: (i, 0, 0, 0))
    oshape = jax.ShapeDtypeStruct((B_TOT, 8, 8, 3), jnp.float32)
    return pl.pallas_call(
        _vqvae_block,
        grid=grid,
        in_specs=[blk, full(w1r), full(b1), full(w2c), full(b2), full(cb),
                  full(cbt), full(d1t), full(db1), full(d2a), full(db2)],
        out_specs=[oblk] * 16,
        out_shape=[oshape] * 16,
        interpret=interpret,
    )(p1p, w1r, b1, w2c, b2, cb, cbt, d1t, db1, d2a, db2)


def kernel(x, enc_w1, enc_b1, enc_w2, enc_b2, codebook,
           dec_w1, dec_b1, dec_w2, dec_b2):
    xn = jnp.transpose(x, (0, 2, 3, 1))
    xp = jnp.pad(xn, ((0, 0), (1, 1), (1, 1), (0, 0)))
    cols = [xp[:, kh:kh + 31:2, kw:kw + 31:2, :]
            for kh in range(4) for kw in range(4)]
    p1 = jnp.concatenate(cols, axis=-1)
    p1p = jnp.stack([p1[:, py::2, px::2, :]
                     for py in range(2) for px in range(2)], axis=1)

    w1r = enc_w1.transpose(2, 3, 1, 0).reshape(48, HID)
    w2k = enc_w2.transpose(2, 3, 1, 0)
    w2c = jnp.stack(
        [jnp.concatenate(
            [w2k[(1 - pr) + 2 * oy, (1 - pc) + 2 * ox]
             for oy in range(2) for ox in range(2)], axis=0)
         for pr in range(2) for pc in range(2)], axis=0)
    d1t = dec_w1.transpose(1, 2, 3, 0).reshape(D, 16 * HID)
    d2k = dec_w2.transpose(2, 3, 1, 0)
    d2a = jnp.stack(
        [jnp.concatenate(
            [d2k[(ty & 1) + 2 * i, (tx & 1) + 2 * j]
             for i in range(2) for j in range(2)], axis=0)
         for ty in range(4) for tx in range(4)], axis=0)

    subs = _run(p1p, w1r, enc_b1, w2c, enc_b2, codebook, codebook.T,
                d1t, dec_b1, d2a, dec_b2)

    rows = [jnp.stack(subs[ty * 4:ty * 4 + 4], axis=3) for ty in range(4)]
    st = jnp.stack(rows, axis=2)
    out = st.reshape(B_TOT, 32, 32, 3)
    return jnp.transpose(out, (0, 3, 1, 2))


# speedup vs baseline: 10.8801x; 1.0123x over previous
import jax
import jax.numpy as jnp
from jax.experimental import pallas as pl

G = 16
B_TOT = 256
K = 1024
D = 64
HID = 128


def _mm(a, b):
    return jax.lax.dot_general(a, b, (((1,), (0,)), ((), ())),
                               preferred_element_type=jnp.float32)


def _vqvae_block(p1_ref, w1_ref, b1_ref, w2c_ref, b2_ref, cb_ref, cbt_ref,
                 d1t_ref, db1_ref, d2a_ref, db2_ref, *out_refs):
    g = p1_ref.shape[0]
    f32 = jnp.float32

    p1 = p1_ref[...]
    m = _mm(p1.reshape(g * 4 * 64, 48), w1_ref[...])
    h1 = jax.nn.relu(m + b1_ref[...][None, :]).reshape(g, 4, 8, 8, HID)

    h1p = {}
    for py in range(2):
        for px in range(2):
            h1p[(py, px)] = jnp.pad(
                h1[:, py * 2 + px],
                ((0, 0), (py, 1 - py), (px, 1 - px), (0, 0)))

    w2c = w2c_ref[...]
    e = jnp.zeros((g * 64, D), f32)
    for pr in range(2):
        for pc in range(2):
            a = h1p[(pr, pc)]
            cat = jnp.concatenate(
                [a[:, oy:oy + 8, ox:ox + 8, :]
                 for oy in range(2) for ox in range(2)], axis=-1)
            e = e + _mm(cat.reshape(g * 64, 4 * HID), w2c[pr * 2 + pc])
    e = e + b2_ref[...][None, :]

    cb = cb_ref[...]
    cbt = cbt_ref[...]
    cn = jnp.sum(cbt * cbt, axis=0)
    scores = cn[None, :] - 2.0 * _mm(e, cbt)
    idx = jnp.argmin(scores, axis=1)
    onehot = (jax.lax.broadcasted_iota(jnp.int32, (g * 64, K), 1)
              == idx[:, None]).astype(f32)
    q = _mm(onehot, cb)

    d1t = d1t_ref[...]
    db1 = db1_ref[...]
    big = _mm(q, d1t)
    pp1 = [jnp.pad(big[:, HID * t:HID * (t + 1)].reshape(g, 8, 8, HID),
                   ((0, 0), (1, 1), (1, 1), (0, 0)))
           for t in range(16)]
    h2 = {}
    for R in range(2):
        for S in range(2):
            acc = None
            for i in range(2):
                sy = R - 1 + i
                for j in range(2):
                    sx = S - 1 + j
                    t = (R + 2 * i) * 4 + (S + 2 * j)
                    sl = pp1[t][:, 1 + sy:9 + sy, 1 + sx:9 + sx, :]
                    acc = sl if acc is None else acc + sl
            h2[(R, S)] = jax.nn.relu(acc + db1[None, None, None, :])

    d2c = d2a_ref[...]
    db2 = db2_ref[...]
    h2p = {k: jnp.pad(v, ((0, 0), (1, 1), (1, 1), (0, 0)))
           for k, v in h2.items()}
    for ty in range(4):
        R, apar = ty & 1, ty >> 1
        for tx in range(4):
            S, bpar = tx & 1, tx >> 1
            parts = []
            for i in range(2):
                cy = apar + R - 1 + i
                ry, sy = cy & 1, cy >> 1
                for j in range(2):
                    cx = bpar + S - 1 + j
                    rx, sx = cx & 1, cx >> 1
                    parts.append(h2p[(ry, rx)][:, 1 + sy:9 + sy,
                                               1 + sx:9 + sx, :])
            cat = jnp.concatenate(parts, axis=-1)
            res = _mm(cat.reshape(g * 64, 4 * HID), d2c[ty * 4 + tx])
            out_refs[ty * 4 + tx][...] = \
                (res + db2[None, :]).reshape(g, 8, 8, 3)


def _run(p1p, w1r, b1, w2c, b2, cb, cbt, d1t, db1, d2a, db2, *,
         interpret=False):
    grid = (B_TOT // G,)
    full = lambda a: pl.BlockSpec(a.shape, lambda i: (0,) * a.ndim)
    blk = pl.BlockSpec((G, 4, 8, 8, 48), lambda i: (i, 0, 0, 0, 0))
    oblk = pl.BlockSpec((G, 8, 8, 3), lambda i: (i, 0, 0, 0))
    oshape = jax.ShapeDtypeStruct((B_TOT, 8, 8, 3), jnp.float32)
    return pl.pallas_call(
        _vqvae_block,
        grid=grid,
        in_specs=[blk, full(w1r), full(b1), full(w2c), full(b2), full(cb),
                  full(cbt), full(d1t), full(db1), full(d2a), full(db2)],
        out_specs=[oblk] * 16,
        out_shape=[oshape] * 16,
        interpret=interpret,
    )(p1p, w1r, b1, w2c, b2, cb, cbt, d1t, db1, d2a, db2)


def kernel(x, enc_w1, enc_b1, enc_w2, enc_b2, codebook,
           dec_w1, dec_b1, dec_w2, dec_b2):
    xn = jnp.transpose(x, (0, 2, 3, 1))
    xp = jnp.pad(xn, ((0, 0), (1, 1), (1, 1), (0, 0)))
    p1p = jnp.stack(
        [jnp.concatenate(
            [xp[:, 2 * py + kh:2 * py + kh + 29:4,
                2 * px + kw:2 * px + kw + 29:4, :]
             for kh in range(4) for kw in range(4)], axis=-1)
         for py in range(2) for px in range(2)], axis=1)

    w1r = enc_w1.transpose(2, 3, 1, 0).reshape(48, HID)
    w2k = enc_w2.transpose(2, 3, 1, 0)
    w2c = jnp.stack(
        [jnp.concatenate(
            [w2k[(1 - pr) + 2 * oy, (1 - pc) + 2 * ox]
             for oy in range(2) for ox in range(2)], axis=0)
         for pr in range(2) for pc in range(2)], axis=0)
    d1t = dec_w1.transpose(1, 2, 3, 0).reshape(D, 16 * HID)
    d2k = dec_w2.transpose(2, 3, 1, 0)
    d2a = jnp.stack(
        [jnp.concatenate(
            [d2k[(ty & 1) + 2 * i, (tx & 1) + 2 * j]
             for i in range(2) for j in range(2)], axis=0)
         for ty in range(4) for tx in range(4)], axis=0)

    subs = _run(p1p, w1r, enc_b1, w2c, enc_b2, codebook, codebook.T,
                d1t, dec_b1, d2a, dec_b2)

    st = jnp.stack(subs, axis=0).reshape(4, 4, B_TOT, 8, 8, 3)
    return st.transpose(2, 5, 3, 0, 4, 1).reshape(B_TOT, 3, 32, 32)


# speedup vs baseline: 13.3846x; 1.2302x over previous
import jax
import jax.numpy as jnp
from jax.experimental import pallas as pl

G = 16
B_TOT = 256
K = 1024
D = 64
HID = 128


def _mm(a, b):
    return jax.lax.dot_general(a, b, (((1,), (0,)), ((), ())),
                               preferred_element_type=jnp.float32)


def _vqvae_block(p1_ref, w1_ref, b1_ref, w2c_ref, b2_ref, cb_ref, cbt_ref,
                 d1t_ref, db1_ref, d2a_ref, db2_ref, out_ref):
    g = p1_ref.shape[0]
    f32 = jnp.float32

    p1 = p1_ref[...]
    m = _mm(p1.reshape(g * 4 * 64, 48), w1_ref[...])
    h1 = jax.nn.relu(m + b1_ref[...][None, :]).reshape(g, 4, 8, 8, HID)

    h1p = {}
    for py in range(2):
        for px in range(2):
            h1p[(py, px)] = jnp.pad(
                h1[:, py * 2 + px],
                ((0, 0), (py, 1 - py), (px, 1 - px), (0, 0)))

    w2c = w2c_ref[...]
    e = jnp.zeros((g * 64, D), f32)
    for pr in range(2):
        for pc in range(2):
            a = h1p[(pr, pc)]
            cat = jnp.concatenate(
                [a[:, oy:oy + 8, ox:ox + 8, :]
                 for oy in range(2) for ox in range(2)], axis=-1)
            e = e + _mm(cat.reshape(g * 64, 4 * HID), w2c[pr * 2 + pc])
    e = e + b2_ref[...][None, :]

    cb = cb_ref[...]
    cbt = cbt_ref[...]
    cn = jnp.sum(cbt * cbt, axis=0)
    scores = cn[None, :] - 2.0 * _mm(e, cbt)
    idx = jnp.argmin(scores, axis=1)
    onehot = (jax.lax.broadcasted_iota(jnp.int32, (g * 64, K), 1)
              == idx[:, None]).astype(f32)
    q = _mm(onehot, cb)

    d1t = d1t_ref[...]
    db1 = db1_ref[...]
    big = _mm(q, d1t)
    pp1 = [jnp.pad(big[:, HID * t:HID * (t + 1)].reshape(g, 8, 8, HID),
                   ((0, 0), (1, 1), (1, 1), (0, 0)))
           for t in range(16)]
    h2 = {}
    for R in range(2):
        for S in range(2):
            acc = None
            for i in range(2):
                sy = R - 1 + i
                for j in range(2):
                    sx = S - 1 + j
                    t = (R + 2 * i) * 4 + (S + 2 * j)
                    sl = pp1[t][:, 1 + sy:9 + sy, 1 + sx:9 + sx, :]
                    acc = sl if acc is None else acc + sl
            h2[(R, S)] = jax.nn.relu(acc + db1[None, None, None, :])

    d2c = d2a_ref[...]
    db2 = db2_ref[...]
    h2p = {k: jnp.pad(v, ((0, 0), (1, 1), (1, 1), (0, 0)))
           for k, v in h2.items()}
    acc48 = None
    for ty in range(4):
        R, apar = ty & 1, ty >> 1
        for tx in range(4):
            S, bpar = tx & 1, tx >> 1
            parts = []
            for i in range(2):
                cy = apar + R - 1 + i
                ry, sy = cy & 1, cy >> 1
                for j in range(2):
                    cx = bpar + S - 1 + j
                    rx, sx = cx & 1, cx >> 1
                    parts.append(h2p[(ry, rx)][:, 1 + sy:9 + sy,
                                               1 + sx:9 + sx, :])
            cat = jnp.concatenate(parts, axis=-1)
            res = _mm(cat.reshape(g * 64, 4 * HID), d2c[ty * 4 + tx])
            acc48 = res if acc48 is None else acc48 + res
    out_ref[...] = (acc48 + db2[None, :]).reshape(g, 8, 8, 48)


def _run(p1p, w1r, b1, w2c, b2, cb, cbt, d1t, db1, d2a, db2, *,
         interpret=False):
    grid = (B_TOT // G,)
    full = lambda a: pl.BlockSpec(a.shape, lambda i: (0,) * a.ndim)
    blk = pl.BlockSpec((G, 4, 8, 8, 48), lambda i: (i, 0, 0, 0, 0))
    oblk = pl.BlockSpec((G, 8, 8, 48), lambda i: (i, 0, 0, 0))
    oshape = jax.ShapeDtypeStruct((B_TOT, 8, 8, 48), jnp.float32)
    return pl.pallas_call(
        _vqvae_block,
        grid=grid,
        in_specs=[blk, full(w1r), full(b1), full(w2c), full(b2), full(cb),
                  full(cbt), full(d1t), full(db1), full(d2a), full(db2)],
        out_specs=oblk,
        out_shape=oshape,
        interpret=interpret,
    )(p1p, w1r, b1, w2c, b2, cb, cbt, d1t, db1, d2a, db2)


def kernel(x, enc_w1, enc_b1, enc_w2, enc_b2, codebook,
           dec_w1, dec_b1, dec_w2, dec_b2):
    xn = jnp.transpose(x, (0, 2, 3, 1))
    xp = jnp.pad(xn, ((0, 0), (1, 1), (1, 1), (0, 0)))
    p1p = jnp.stack(
        [jnp.concatenate(
            [xp[:, 2 * py + kh:2 * py + kh + 29:4,
                2 * px + kw:2 * px + kw + 29:4, :]
             for kh in range(4) for kw in range(4)], axis=-1)
         for py in range(2) for px in range(2)], axis=1)

    w1r = enc_w1.transpose(2, 3, 1, 0).reshape(48, HID)
    w2k = enc_w2.transpose(2, 3, 1, 0)
    w2c = jnp.stack(
        [jnp.concatenate(
            [w2k[(1 - pr) + 2 * oy, (1 - pc) + 2 * ox]
             for oy in range(2) for ox in range(2)], axis=0)
         for pr in range(2) for pc in range(2)], axis=0)
    d1t = dec_w1.transpose(1, 2, 3, 0).reshape(D, 16 * HID)
    d2k = dec_w2.transpose(2, 3, 1, 0)
    d2a = jnp.stack(
        [jnp.pad(
            jnp.concatenate(
                [d2k[(ty & 1) + 2 * i, (tx & 1) + 2 * j]
                 for i in range(2) for j in range(2)], axis=0),
            ((0, 0), (3 * (ty * 4 + tx), 45 - 3 * (ty * 4 + tx))))
         for ty in range(4) for tx in range(4)], axis=0)
    db2t = jnp.tile(dec_b2, 16)

    res48 = _run(p1p, w1r, enc_b1, w2c, enc_b2, codebook, codebook.T,
                 d1t, dec_b1, d2a, db2t)

    st = res48.reshape(B_TOT, 8, 8, 4, 4, 3)
    return st.transpose(0, 5, 1, 3, 2, 4).reshape(B_TOT, 3, 32, 32)
